# baseline (speedup 1.0000x reference)
import jax
import jax.numpy as jnp
from jax.experimental import pallas as pl
from jax.experimental.pallas import tpu as pltpu

N = 10000
BM = 200
_ADJ_PREC = jax.lax.Precision.DEFAULT
_PREC = jax.lax.Precision.HIGHEST


def _dot(a, b, precision=_PREC):
    return jnp.dot(a, b, precision=precision,
                   preferred_element_type=jnp.float32)


def _prep_body(x_ref, wfc_ref, bfc_ref, w0_ref, g_ref):
    h = _dot(x_ref[...], wfc_ref[...]) + bfc_ref[...]
    g_ref[...] = _dot(h, w0_ref[...])


def _pass1_body(g_ref, b_ref, adj_ref, out_ref):
    out_ref[...] = jnp.maximum(
        _dot(adj_ref[...], g_ref[...], _ADJ_PREC) + b_ref[...], 0.0)


def _pass2_body(h_ref, w_ref, b_ref, adj_ref, out_ref, g_ref):
    @pl.when(pl.program_id(0) == 0)
    def _():
        g_ref[...] = _dot(h_ref[...], w_ref[...])

    out_ref[...] = jnp.maximum(
        _dot(adj_ref[...], g_ref[...], _ADJ_PREC) + b_ref[...], 0.0)


def _pass3_body(h_ref, w_ref, b_ref, adj_ref, logits_ref, pred_ref, g_ref):
    @pl.when(pl.program_id(0) == 0)
    def _():
        g_ref[...] = _dot(h_ref[...], w_ref[...])

    logits = _dot(adj_ref[...], g_ref[...], _ADJ_PREC) + b_ref[...]
    logits_ref[...] = logits
    nclass = logits.shape[1]
    idx = jax.lax.broadcasted_iota(jnp.int32, logits.shape, 1)
    maxv = jnp.max(logits, axis=1, keepdims=True)
    pred_ref[...] = jnp.min(jnp.where(logits == maxv, idx, nclass), axis=1,
                            keepdims=True)


def _const_spec(shape):
    return pl.BlockSpec(shape, lambda i: (0,) * len(shape))


def kernel(x, adj, W_fc, b_fc, W0, b0, W1, b1, W_exit, b_exit):
    n, nfeat = x.shape
    nhid = W0.shape[0]
    nclass = W_exit.shape[1]
    grid = (n // BM,)

    adj_spec = pl.BlockSpec((BM, n), lambda i: (i, 0))

    g0 = pl.pallas_call(
        _prep_body,
        grid=(10,),
        in_specs=[
            pl.BlockSpec((n // 10, nfeat), lambda i: (i, 0)),
            _const_spec((nfeat, nhid)),
            _const_spec((1, nhid)),
            _const_spec((nhid, nhid)),
        ],
        out_specs=pl.BlockSpec((n // 10, nhid), lambda i: (i, 0)),
        out_shape=jax.ShapeDtypeStruct((n, nhid), jnp.float32),
    )(x, W_fc, b_fc.reshape(1, nhid), W0)

    h1 = pl.pallas_call(
        _pass1_body,
        grid=grid,
        in_specs=[
            _const_spec((n, nhid)),
            _const_spec((1, nhid)),
            adj_spec,
        ],
        out_specs=pl.BlockSpec((BM, nhid), lambda i: (i, 0)),
        out_shape=jax.ShapeDtypeStruct((n, nhid), jnp.float32),
    )(g0, b0.reshape(1, nhid), adj)

    h2 = pl.pallas_call(
        _pass2_body,
        grid=grid,
        in_specs=[
            _const_spec((n, nhid)),
            _const_spec((nhid, nhid)),
            _const_spec((1, nhid)),
            adj_spec,
        ],
        out_specs=pl.BlockSpec((BM, nhid), lambda i: (i, 0)),
        out_shape=jax.ShapeDtypeStruct((n, nhid), jnp.float32),
        scratch_shapes=[pltpu.VMEM((n, nhid), jnp.float32)],
    )(h1, W1, b1.reshape(1, nhid), adj)

    logits, pred2 = pl.pallas_call(
        _pass3_body,
        grid=grid,
        in_specs=[
            _const_spec((n, nhid)),
            _const_spec((nhid, nclass)),
            _const_spec((1, nclass)),
            adj_spec,
        ],
        out_specs=[
            pl.BlockSpec((BM, nclass), lambda i: (i, 0)),
            pl.BlockSpec((BM, 1), lambda i: (i, 0)),
        ],
        out_shape=[
            jax.ShapeDtypeStruct((n, nclass), jnp.float32),
            jax.ShapeDtypeStruct((n, 1), jnp.int32),
        ],
        scratch_shapes=[pltpu.VMEM((n, nclass), jnp.float32)],
    )(h2, W_exit, b_exit.reshape(1, nclass), adj)

    return (logits, pred2.reshape(n))
